# baseline (speedup 1.0000x reference)
import functools

import jax
import jax.numpy as jnp
from jax import lax
from jax.experimental import pallas as pl
from jax.experimental.pallas import tpu as pltpu
from jax.experimental.pallas import tpu_sc as plsc

_D = 1024
_E = 8
_N = 4096
_A = 2 * _N
_BLK = 128
_M = _A + _E * _BLK
_NB = _M // _BLK
_NBPAD = 80
_GROWS = 512


def _gate_body(x_ref, wg_ref, bg_ref, lng_ref, lnb_ref,
               xn_ref, e1_ref, e2_ref, p1_ref, p2_ref, gs_ref, st_ref,
               cnt_ref):
    i = pl.program_id(0)
    x = x_ref[...]
    mu = jnp.mean(x, axis=1, keepdims=True)
    xc = x - mu
    var = jnp.mean(xc * xc, axis=1, keepdims=True)
    xn_ref[...] = xc * lax.rsqrt(var + 1e-5) * lng_ref[...] + lnb_ref[...]

    z = jnp.dot(x, wg_ref[...], preferred_element_type=jnp.float32) + bg_ref[...]
    lg = jnp.where(z >= 0, z, 0.01 * z)

    iot = lax.broadcasted_iota(jnp.int32, (_GROWS, _E), 1)
    m1 = jnp.max(lg, axis=1, keepdims=True)
    i1 = jnp.min(jnp.where(lg == m1, iot, _E), axis=1, keepdims=True)
    masked = jnp.where(iot == i1, -jnp.inf, lg)
    m2 = jnp.max(masked, axis=1, keepdims=True)
    i2 = jnp.min(jnp.where(masked == m2, iot, _E), axis=1, keepdims=True)
    t = jnp.exp(m2 - m1)
    p1 = 1.0 / (1.0 + t)
    p2 = t * p1

    e1_ref[...] = i1
    e2_ref[...] = i2
    p1_ref[...] = p1
    p2_ref[...] = p2

    g1 = jnp.where(iot == i1, p1, 0.0)
    g2 = jnp.where(iot == i2, p2, 0.0)
    gsp = jnp.sum(g1 + g2, axis=0).reshape(1, _E)

    iot16 = lax.broadcasted_iota(jnp.int32, (128, 16), 1)
    rows = []
    for s in range(_GROWS // 128):
        a = i1[s * 128:(s + 1) * 128]
        b = i2[s * 128:(s + 1) * 128]
        c = ((iot16 == a).astype(jnp.int32) + (iot16 == b).astype(jnp.int32))
        rows.append(jnp.sum(c, axis=0).reshape(1, 16))
    cnt_ref[...] = jnp.concatenate(rows, axis=0).reshape(1, 4, 16)

    zp = jnp.sum(lg * lg)
    s1p = jnp.sum(p1) + jnp.sum(p2)
    s2p = jnp.sum(p1 * p1) + jnp.sum(p2 * p2)
    li = lax.broadcasted_iota(jnp.int32, (1, _E), 1)
    strow = (jnp.where(li == 0, zp, 0.0) + jnp.where(li == 1, s1p, 0.0)
             + jnp.where(li == 2, s2p, 0.0))

    @pl.when(i == 0)
    def _():
        gs_ref[...] = gsp
        st_ref[...] = strow

    @pl.when(i > 0)
    def _():
        gs_ref[...] += gsp
        st_ref[...] += strow


_gating = pl.pallas_call(
    _gate_body,
    grid=(_N // _GROWS,),
    in_specs=[
        pl.BlockSpec((_GROWS, _D), lambda i: (i, 0)),
        pl.BlockSpec((_D, _E), lambda i: (0, 0)),
        pl.BlockSpec((1, _E), lambda i: (0, 0)),
        pl.BlockSpec((1, _D), lambda i: (0, 0)),
        pl.BlockSpec((1, _D), lambda i: (0, 0)),
    ],
    out_specs=[
        pl.BlockSpec((_GROWS, _D), lambda i: (i, 0)),
        pl.BlockSpec((_GROWS, 1), lambda i: (i, 0)),
        pl.BlockSpec((_GROWS, 1), lambda i: (i, 0)),
        pl.BlockSpec((_GROWS, 1), lambda i: (i, 0)),
        pl.BlockSpec((_GROWS, 1), lambda i: (i, 0)),
        pl.BlockSpec((1, _E), lambda i: (0, 0)),
        pl.BlockSpec((1, _E), lambda i: (0, 0)),
        pl.BlockSpec((1, 4, 16), lambda i: (i, 0, 0)),
    ],
    out_shape=[
        jax.ShapeDtypeStruct((_N, _D), jnp.float32),
        jax.ShapeDtypeStruct((_N, 1), jnp.int32),
        jax.ShapeDtypeStruct((_N, 1), jnp.int32),
        jax.ShapeDtypeStruct((_N, 1), jnp.float32),
        jax.ShapeDtypeStruct((_N, 1), jnp.float32),
        jax.ShapeDtypeStruct((1, _E), jnp.float32),
        jax.ShapeDtypeStruct((1, _E), jnp.float32),
        jax.ShapeDtypeStruct((_N // _GROWS, 4, 16), jnp.int32),
    ],
    compiler_params=pltpu.CompilerParams(dimension_semantics=("arbitrary",)),
)


def _dispatch_body(xn_hbm, e1_hbm, e2_hbm, cnt_hbm,
                   xs_hbm, dest_hbm, be_hbm,
                   e1_v, e2_v, f_v, dest_v, base_v, off_v,
                   cnt_v, idx0_v, idx1_v, xrow_v, be_v, sem_ld, sem_st):
    w = lax.axis_index("s") * 2 + lax.axis_index("c")
    iota = lax.iota(jnp.int32, 16)
    tok = _N // 32
    na = 2 * tok

    pltpu.sync_copy(e1_hbm.at[pl.ds(w * tok, tok)], e1_v)
    pltpu.sync_copy(e2_hbm.at[pl.ds(w * tok, tok)], e2_v)
    pltpu.sync_copy(cnt_hbm, cnt_v)

    totals = jnp.zeros((16,), jnp.int32)
    prev = jnp.zeros((16,), jnp.int32)
    for r in range(32):
        row = cnt_v[pl.ds(r * 16, 16)]
        totals = totals + row
        prev = prev + jnp.where(r < w, row, 0)
    padded = ((totals + (_BLK - 1)) // _BLK) * _BLK
    incl = plsc.cumsum(padded)
    off = incl - padded
    off_v[...] = off
    base_v[...] = off + prev

    for s in range(tok // 16):
        v1 = e1_v[pl.ds(s * 16, 16)]
        v2 = e2_v[pl.ds(s * 16, 16)]
        plsc.store_scatter(f_v, [32 * s + 2 * iota], v1)
        plsc.store_scatter(f_v, [32 * s + 2 * iota + 1], v2)

    cnt = [jnp.zeros((), jnp.int32) for _ in range(_E)]
    for s in range(na // 16):
        v = f_v[pl.ds(s * 16, 16)]
        b = plsc.load_gather(base_v, [v])
        d = jnp.zeros((16,), jnp.int32)
        for e in range(_E):
            m = v == e
            mi = m.astype(jnp.int32)
            c = plsc.cumsum(mi)
            d = jnp.where(m, c - 1 + cnt[e], d)
            cnt[e] = cnt[e] + jnp.sum(mi)
        dest_v[pl.ds(s * 16, 16)] = b + d
    pltpu.sync_copy(dest_v, dest_hbm.at[pl.ds(w * na, na)])

    @pl.when(w == 0)
    def _():
        for c in range(_NBPAD // 16):
            bb = (iota + c * 16) * _BLK
            cntv = jnp.zeros((16,), jnp.int32)
            for e in range(_E):
                oe = plsc.load_gather(off_v, [jnp.full((16,), e, jnp.int32)])
                cntv = cntv + (bb >= oe).astype(jnp.int32)
            be_v[pl.ds(c * 16, 16)] = cntv - 1
        pltpu.sync_copy(be_v, be_hbm)

    ch_tok = 32
    n_ch = tok // ch_tok
    for ch in range(n_ch):
        for s in range(ch_tok // 16):
            li = ch * 2 * ch_tok + 2 * (s * 16 + iota)
            idx0_v[ch, pl.ds(s * 16, 16)] = plsc.load_gather(dest_v, [li])
            idx1_v[ch, pl.ds(s * 16, 16)] = plsc.load_gather(dest_v, [li + 1])

    loads = [None] * n_ch
    stores = [None] * n_ch
    loads[0] = pltpu.async_copy(
        xn_hbm.at[pl.ds(w * tok, ch_tok)], xrow_v.at[0], sem_ld)
    for ch in range(n_ch):
        loads[ch].wait()
        if ch + 1 < n_ch:
            if ch >= 1:
                stores[ch - 1][0].wait()
                stores[ch - 1][1].wait()
            loads[ch + 1] = pltpu.async_copy(
                xn_hbm.at[pl.ds(w * tok + (ch + 1) * ch_tok, ch_tok)],
                xrow_v.at[(ch + 1) % 2], sem_ld)
        stores[ch] = (
            pltpu.async_copy(xrow_v.at[ch % 2], xs_hbm.at[idx0_v.at[ch]],
                             sem_st),
            pltpu.async_copy(xrow_v.at[ch % 2], xs_hbm.at[idx1_v.at[ch]],
                             sem_st),
        )
    stores[n_ch - 2][0].wait()
    stores[n_ch - 2][1].wait()
    stores[n_ch - 1][0].wait()
    stores[n_ch - 1][1].wait()


_dispatch = pl.kernel(
    _dispatch_body,
    out_type=[
        jax.ShapeDtypeStruct((_M, _D), jnp.float32),
        jax.ShapeDtypeStruct((_A,), jnp.int32),
        jax.ShapeDtypeStruct((_NBPAD,), jnp.int32),
    ],
    mesh=plsc.VectorSubcoreMesh(core_axis_name="c", subcore_axis_name="s",
                                num_cores=2, num_subcores=16),
    scratch_types=[
        pltpu.VMEM((_N // 32,), jnp.int32),
        pltpu.VMEM((_N // 32,), jnp.int32),
        pltpu.VMEM((_A // 32,), jnp.int32),
        pltpu.VMEM((_A // 32,), jnp.int32),
        pltpu.VMEM((16,), jnp.int32),
        pltpu.VMEM((16,), jnp.int32),
        pltpu.VMEM((512,), jnp.int32),
        pltpu.VMEM((4, 32), jnp.int32),
        pltpu.VMEM((4, 32), jnp.int32),
        pltpu.VMEM((2, 32, _D), jnp.float32),
        pltpu.VMEM((_NBPAD,), jnp.int32),
        pltpu.SemaphoreType.DMA,
        pltpu.SemaphoreType.DMA,
    ],
    compiler_params=pltpu.CompilerParams(needs_layout_passes=False),
)


def _ffn_inner(x_ref, w1_ref, b1_ref, w2_ref, b2_ref, o_ref):
    h = jnp.dot(x_ref[...], w1_ref[0], preferred_element_type=jnp.float32)
    h = jnp.maximum(h + b1_ref[0], 0.0)
    o_ref[...] = (jnp.dot(h, w2_ref[0], preferred_element_type=jnp.float32)
                  + b2_ref[0])


def _ffn_outer(be_ref, x_hbm, w1_hbm, b1_hbm, w2_hbm, b2_hbm, o_hbm):
    wspec = lambda shape, imap: pl.BlockSpec(
        shape, imap,
        pipeline_mode=pl.Buffered(buffer_count=2, use_lookahead=True))
    pipe = pltpu.emit_pipeline(
        _ffn_inner,
        grid=(_NB,),
        in_specs=[
            pl.BlockSpec((_BLK, _D), lambda i: (i, 0)),
            wspec((1, _D, 2 * _D), lambda i: (be_ref[i], 0, 0)),
            wspec((1, 1, 2 * _D), lambda i: (be_ref[i], 0, 0)),
            wspec((1, 2 * _D, _D), lambda i: (be_ref[i], 0, 0)),
            wspec((1, 1, _D), lambda i: (be_ref[i], 0, 0)),
        ],
        out_specs=[pl.BlockSpec((_BLK, _D), lambda i: (i, 0))],
    )
    pipe(x_hbm, w1_hbm, b1_hbm, w2_hbm, b2_hbm, o_hbm)


_ffn = pl.pallas_call(
    _ffn_outer,
    grid_spec=pltpu.PrefetchScalarGridSpec(
        num_scalar_prefetch=1,
        grid=(),
        in_specs=[pl.BlockSpec(memory_space=pltpu.HBM)] * 5,
        out_specs=pl.BlockSpec(memory_space=pltpu.HBM),
    ),
    out_shape=jax.ShapeDtypeStruct((_M, _D), jnp.float32),
)


def _combine_body(x_hbm, o_hbm, dest_hbm, p1_hbm, p2_hbm, y_hbm,
                  dvec_v, idx0_v, idx1_v, xr_v, o0_v, o1_v, p1_v, p2_v,
                  sem_x, sem_g0, sem_g1, sem_y):
    w = lax.axis_index("s") * 2 + lax.axis_index("c")
    iota = lax.iota(jnp.int32, 16)
    tok = _N // 32
    ch_tok = 16
    n_ch = tok // ch_tok

    pltpu.sync_copy(dest_hbm.at[pl.ds(w * tok * 2, 2 * tok)], dvec_v)
    pltpu.sync_copy(p1_hbm.at[pl.ds(w * tok, tok)], p1_v)
    pltpu.sync_copy(p2_hbm.at[pl.ds(w * tok, tok)], p2_v)
    for ch in range(n_ch):
        li = 2 * (ch * 16 + iota)
        idx0_v[ch, :] = plsc.load_gather(dvec_v, [li])
        idx1_v[ch, :] = plsc.load_gather(dvec_v, [li + 1])

    def start(ch):
        b = ch % 2
        t0 = w * tok + ch * ch_tok
        return (
            pltpu.async_copy(x_hbm.at[pl.ds(t0, ch_tok)], xr_v.at[b], sem_x),
            pltpu.async_copy(o_hbm.at[idx0_v.at[ch]], o0_v.at[b], sem_g0),
            pltpu.async_copy(o_hbm.at[idx1_v.at[ch]], o1_v.at[b], sem_g1),
        )

    pend = [None] * n_ch
    st = [None] * n_ch
    pend[0] = start(0)
    for ch in range(n_ch):
        b = ch % 2
        for dsc in pend[ch]:
            dsc.wait()
        if ch + 1 < n_ch:
            if ch >= 1:
                st[ch - 1].wait()
            pend[ch + 1] = start(ch + 1)
        for i in range(ch_tok):
            bi = jnp.full((16,), ch * ch_tok + i, jnp.int32)
            pv1 = plsc.load_gather(p1_v, [bi])
            pv2 = plsc.load_gather(p2_v, [bi])

            @plsc.parallel_loop(0, _D // 16, unroll=8)
            def _(j, b=b, i=i, pv1=pv1, pv2=pv2):
                sl = pl.ds(j * 16, 16)
                xr_v[b, i, sl] = (xr_v[b, i, sl] + pv1 * o0_v[b, i, sl]
                                  + pv2 * o1_v[b, i, sl])
        st[ch] = pltpu.async_copy(
            xr_v.at[b], y_hbm.at[pl.ds(w * tok + ch * ch_tok, ch_tok)],
            sem_y)
    st[n_ch - 2].wait()
    st[n_ch - 1].wait()


_combine = pl.kernel(
    _combine_body,
    out_type=jax.ShapeDtypeStruct((_N, _D), jnp.float32),
    mesh=plsc.VectorSubcoreMesh(core_axis_name="c", subcore_axis_name="s",
                                num_cores=2, num_subcores=16),
    scratch_types=[
        pltpu.VMEM((256,), jnp.int32),
        pltpu.VMEM((8, 16), jnp.int32),
        pltpu.VMEM((8, 16), jnp.int32),
        pltpu.VMEM((2, 16, _D), jnp.float32),
        pltpu.VMEM((2, 16, _D), jnp.float32),
        pltpu.VMEM((2, 16, _D), jnp.float32),
        pltpu.VMEM((128,), jnp.float32),
        pltpu.VMEM((128,), jnp.float32),
        pltpu.SemaphoreType.DMA,
        pltpu.SemaphoreType.DMA,
        pltpu.SemaphoreType.DMA,
        pltpu.SemaphoreType.DMA,
    ],
    compiler_params=pltpu.CompilerParams(needs_layout_passes=False),
)


def kernel(x, Wg, bg, ln_g, ln_b, W1, b1, W2, b2):
    b, l, d = x.shape
    xf = x.reshape(_N, _D)
    xn, e1, e2, p1, p2, gs, st, cnts = _gating(
        xf, Wg, bg.reshape(1, _E), ln_g.reshape(1, _D), ln_b.reshape(1, _D))
    xs, dest, be = _dispatch(xn, e1.reshape(_N), e2.reshape(_N),
                             cnts.reshape(32 * 16))
    o = _ffn(be, xs, W1, b1.reshape(_E, 1, 2 * _D), W2,
             b2.reshape(_E, 1, _D))
    y = _combine(xf, o, dest, p1.reshape(_N), p2.reshape(_N))

    zloss = st[0, 0] / (_N * _E)
    s1 = st[0, 1]
    s2 = st[0, 2]
    ne = _N * _E
    mean = s1 / ne
    var = (s2 - s1 * s1 / ne) / (ne - 1)
    cvloss = var / (mean * mean + 1e-10)
    return y.reshape(b, l, d), gs.reshape(_E), zloss, cvloss


# speedup vs baseline: 1.1262x; 1.1262x over previous
import functools

import jax
import jax.numpy as jnp
from jax import lax
from jax.experimental import pallas as pl
from jax.experimental.pallas import tpu as pltpu
from jax.experimental.pallas import tpu_sc as plsc

_D = 1024
_E = 8
_N = 4096
_A = 2 * _N
_BLK = 256
_M = _A + _E * _BLK
_NB = _M // _BLK
_NBPAD = 64
_GROWS = 512


def _gate_body(x_ref, wg_ref, bg_ref, lng_ref, lnb_ref,
               xn_ref, e1_ref, e2_ref, p1_ref, p2_ref, gs_ref, st_ref,
               cnt_ref):
    i = pl.program_id(0)
    x = x_ref[...]
    mu = jnp.mean(x, axis=1, keepdims=True)
    xc = x - mu
    var = jnp.mean(xc * xc, axis=1, keepdims=True)
    xn_ref[...] = xc * lax.rsqrt(var + 1e-5) * lng_ref[...] + lnb_ref[...]

    z = jnp.dot(x, wg_ref[...], preferred_element_type=jnp.float32) + bg_ref[...]
    lg = jnp.where(z >= 0, z, 0.01 * z)

    iot = lax.broadcasted_iota(jnp.int32, (_GROWS, _E), 1)
    m1 = jnp.max(lg, axis=1, keepdims=True)
    i1 = jnp.min(jnp.where(lg == m1, iot, _E), axis=1, keepdims=True)
    masked = jnp.where(iot == i1, -jnp.inf, lg)
    m2 = jnp.max(masked, axis=1, keepdims=True)
    i2 = jnp.min(jnp.where(masked == m2, iot, _E), axis=1, keepdims=True)
    t = jnp.exp(m2 - m1)
    p1 = 1.0 / (1.0 + t)
    p2 = t * p1

    e1_ref[...] = i1
    e2_ref[...] = i2
    p1_ref[...] = p1
    p2_ref[...] = p2

    g1 = jnp.where(iot == i1, p1, 0.0)
    g2 = jnp.where(iot == i2, p2, 0.0)
    gsp = jnp.sum(g1 + g2, axis=0).reshape(1, _E)

    iot16 = lax.broadcasted_iota(jnp.int32, (128, 16), 1)
    rows = []
    for s in range(_GROWS // 128):
        a = i1[s * 128:(s + 1) * 128]
        b = i2[s * 128:(s + 1) * 128]
        c = ((iot16 == a).astype(jnp.int32) + (iot16 == b).astype(jnp.int32))
        rows.append(jnp.sum(c, axis=0).reshape(1, 16))
    cnt_ref[...] = jnp.concatenate(rows, axis=0).reshape(1, 4, 16)

    zp = jnp.sum(lg * lg)
    s1p = jnp.sum(p1) + jnp.sum(p2)
    s2p = jnp.sum(p1 * p1) + jnp.sum(p2 * p2)
    li = lax.broadcasted_iota(jnp.int32, (1, _E), 1)
    strow = (jnp.where(li == 0, zp, 0.0) + jnp.where(li == 1, s1p, 0.0)
             + jnp.where(li == 2, s2p, 0.0))

    @pl.when(i == 0)
    def _():
        gs_ref[...] = gsp
        st_ref[...] = strow

    @pl.when(i > 0)
    def _():
        gs_ref[...] += gsp
        st_ref[...] += strow


_gating = pl.pallas_call(
    _gate_body,
    grid=(_N // _GROWS,),
    in_specs=[
        pl.BlockSpec((_GROWS, _D), lambda i: (i, 0)),
        pl.BlockSpec((_D, _E), lambda i: (0, 0)),
        pl.BlockSpec((1, _E), lambda i: (0, 0)),
        pl.BlockSpec((1, _D), lambda i: (0, 0)),
        pl.BlockSpec((1, _D), lambda i: (0, 0)),
    ],
    out_specs=[
        pl.BlockSpec((_GROWS, _D), lambda i: (i, 0)),
        pl.BlockSpec((_GROWS, 1), lambda i: (i, 0)),
        pl.BlockSpec((_GROWS, 1), lambda i: (i, 0)),
        pl.BlockSpec((_GROWS, 1), lambda i: (i, 0)),
        pl.BlockSpec((_GROWS, 1), lambda i: (i, 0)),
        pl.BlockSpec((1, _E), lambda i: (0, 0)),
        pl.BlockSpec((1, _E), lambda i: (0, 0)),
        pl.BlockSpec((1, 4, 16), lambda i: (i, 0, 0)),
    ],
    out_shape=[
        jax.ShapeDtypeStruct((_N, _D), jnp.float32),
        jax.ShapeDtypeStruct((_N, 1), jnp.int32),
        jax.ShapeDtypeStruct((_N, 1), jnp.int32),
        jax.ShapeDtypeStruct((_N, 1), jnp.float32),
        jax.ShapeDtypeStruct((_N, 1), jnp.float32),
        jax.ShapeDtypeStruct((1, _E), jnp.float32),
        jax.ShapeDtypeStruct((1, _E), jnp.float32),
        jax.ShapeDtypeStruct((_N // _GROWS, 4, 16), jnp.int32),
    ],
    compiler_params=pltpu.CompilerParams(dimension_semantics=("arbitrary",)),
)


def _dispatch_body(xn_hbm, e1_hbm, e2_hbm, cnt_hbm,
                   xs_hbm, dest_hbm, be_hbm,
                   e1_v, e2_v, f_v, dest_v, base_v, off_v,
                   cnt_v, idx0_v, idx1_v, xrow_v, be_v, sem_ld, sem_st):
    w = lax.axis_index("s") * 2 + lax.axis_index("c")
    iota = lax.iota(jnp.int32, 16)
    tok = _N // 32
    na = 2 * tok

    pltpu.sync_copy(e1_hbm.at[pl.ds(w * tok, tok)], e1_v)
    pltpu.sync_copy(e2_hbm.at[pl.ds(w * tok, tok)], e2_v)
    pltpu.sync_copy(cnt_hbm, cnt_v)

    totals = jnp.zeros((16,), jnp.int32)
    prev = jnp.zeros((16,), jnp.int32)
    for r in range(32):
        row = cnt_v[pl.ds(r * 16, 16)]
        totals = totals + row
        prev = prev + jnp.where(r < w, row, 0)
    padded = ((totals + (_BLK - 1)) // _BLK) * _BLK
    incl = plsc.cumsum(padded)
    off = incl - padded
    off_v[...] = off
    base_v[...] = off + prev

    for s in range(tok // 16):
        v1 = e1_v[pl.ds(s * 16, 16)]
        v2 = e2_v[pl.ds(s * 16, 16)]
        plsc.store_scatter(f_v, [32 * s + 2 * iota], v1)
        plsc.store_scatter(f_v, [32 * s + 2 * iota + 1], v2)

    cnt = [jnp.zeros((), jnp.int32) for _ in range(_E)]
    for s in range(na // 16):
        v = f_v[pl.ds(s * 16, 16)]
        b = plsc.load_gather(base_v, [v])
        d = jnp.zeros((16,), jnp.int32)
        for e in range(_E):
            m = v == e
            mi = m.astype(jnp.int32)
            c = plsc.cumsum(mi)
            d = jnp.where(m, c - 1 + cnt[e], d)
            cnt[e] = cnt[e] + jnp.sum(mi)
        dest_v[pl.ds(s * 16, 16)] = b + d
    pltpu.sync_copy(dest_v, dest_hbm.at[pl.ds(w * na, na)])

    @pl.when(w == 0)
    def _():
        for c in range(_NBPAD // 16):
            bb = (iota + c * 16) * _BLK
            cntv = jnp.zeros((16,), jnp.int32)
            for e in range(_E):
                oe = plsc.load_gather(off_v, [jnp.full((16,), e, jnp.int32)])
                cntv = cntv + (bb >= oe).astype(jnp.int32)
            be_v[pl.ds(c * 16, 16)] = cntv - 1
        pltpu.sync_copy(be_v, be_hbm)

    ch_tok = 32
    n_ch = tok // ch_tok
    for ch in range(n_ch):
        for s in range(ch_tok // 16):
            li = ch * 2 * ch_tok + 2 * (s * 16 + iota)
            idx0_v[ch, pl.ds(s * 16, 16)] = plsc.load_gather(dest_v, [li])
            idx1_v[ch, pl.ds(s * 16, 16)] = plsc.load_gather(dest_v, [li + 1])

    loads = [None] * n_ch
    stores = [None] * n_ch
    loads[0] = pltpu.async_copy(
        xn_hbm.at[pl.ds(w * tok, ch_tok)], xrow_v.at[0], sem_ld)
    for ch in range(n_ch):
        loads[ch].wait()
        if ch + 1 < n_ch:
            if ch >= 1:
                stores[ch - 1][0].wait()
                stores[ch - 1][1].wait()
            loads[ch + 1] = pltpu.async_copy(
                xn_hbm.at[pl.ds(w * tok + (ch + 1) * ch_tok, ch_tok)],
                xrow_v.at[(ch + 1) % 2], sem_ld)
        stores[ch] = (
            pltpu.async_copy(xrow_v.at[ch % 2], xs_hbm.at[idx0_v.at[ch]],
                             sem_st),
            pltpu.async_copy(xrow_v.at[ch % 2], xs_hbm.at[idx1_v.at[ch]],
                             sem_st),
        )
    stores[n_ch - 2][0].wait()
    stores[n_ch - 2][1].wait()
    stores[n_ch - 1][0].wait()
    stores[n_ch - 1][1].wait()


_dispatch = pl.kernel(
    _dispatch_body,
    out_type=[
        jax.ShapeDtypeStruct((_M, _D), jnp.float32),
        jax.ShapeDtypeStruct((_A,), jnp.int32),
        jax.ShapeDtypeStruct((_NBPAD,), jnp.int32),
    ],
    mesh=plsc.VectorSubcoreMesh(core_axis_name="c", subcore_axis_name="s",
                                num_cores=2, num_subcores=16),
    scratch_types=[
        pltpu.VMEM((_N // 32,), jnp.int32),
        pltpu.VMEM((_N // 32,), jnp.int32),
        pltpu.VMEM((_A // 32,), jnp.int32),
        pltpu.VMEM((_A // 32,), jnp.int32),
        pltpu.VMEM((16,), jnp.int32),
        pltpu.VMEM((16,), jnp.int32),
        pltpu.VMEM((512,), jnp.int32),
        pltpu.VMEM((4, 32), jnp.int32),
        pltpu.VMEM((4, 32), jnp.int32),
        pltpu.VMEM((2, 32, _D), jnp.float32),
        pltpu.VMEM((_NBPAD,), jnp.int32),
        pltpu.SemaphoreType.DMA,
        pltpu.SemaphoreType.DMA,
    ],
    compiler_params=pltpu.CompilerParams(needs_layout_passes=False),
)


def _ffn_inner(x_ref, w1_ref, b1_ref, w2_ref, b2_ref, o_ref):
    h = jnp.dot(x_ref[...], w1_ref[0], preferred_element_type=jnp.float32)
    h = jnp.maximum(h + b1_ref[0], 0.0)
    o_ref[...] = (jnp.dot(h, w2_ref[0], preferred_element_type=jnp.float32)
                  + b2_ref[0])


def _ffn_outer(be_ref, x_hbm, w1_hbm, b1_hbm, w2_hbm, b2_hbm, o_hbm):
    wspec = lambda shape, imap: pl.BlockSpec(
        shape, imap,
        pipeline_mode=pl.Buffered(buffer_count=2, use_lookahead=True))
    pipe = pltpu.emit_pipeline(
        _ffn_inner,
        grid=(_NB,),
        in_specs=[
            pl.BlockSpec((_BLK, _D), lambda i: (i, 0)),
            wspec((1, _D, 2 * _D), lambda i: (be_ref[i], 0, 0)),
            wspec((1, 1, 2 * _D), lambda i: (be_ref[i], 0, 0)),
            wspec((1, 2 * _D, _D), lambda i: (be_ref[i], 0, 0)),
            wspec((1, 1, _D), lambda i: (be_ref[i], 0, 0)),
        ],
        out_specs=[pl.BlockSpec((_BLK, _D), lambda i: (i, 0))],
    )
    pipe(x_hbm, w1_hbm, b1_hbm, w2_hbm, b2_hbm, o_hbm)


_ffn = pl.pallas_call(
    _ffn_outer,
    grid_spec=pltpu.PrefetchScalarGridSpec(
        num_scalar_prefetch=1,
        grid=(),
        in_specs=[pl.BlockSpec(memory_space=pltpu.HBM)] * 5,
        out_specs=pl.BlockSpec(memory_space=pltpu.HBM),
    ),
    out_shape=jax.ShapeDtypeStruct((_M, _D), jnp.float32),
)


def _combine_body(x_hbm, o_hbm, dest_hbm, p1_hbm, p2_hbm, y_hbm,
                  dvec_v, idx0_v, idx1_v, xr_v, o0_v, o1_v, p1_v, p2_v,
                  sem_x, sem_g0, sem_g1, sem_y):
    w = lax.axis_index("s") * 2 + lax.axis_index("c")
    iota = lax.iota(jnp.int32, 16)
    tok = _N // 32
    ch_tok = 16
    n_ch = tok // ch_tok

    pltpu.sync_copy(dest_hbm.at[pl.ds(w * tok * 2, 2 * tok)], dvec_v)
    pltpu.sync_copy(p1_hbm.at[pl.ds(w * tok, tok)], p1_v)
    pltpu.sync_copy(p2_hbm.at[pl.ds(w * tok, tok)], p2_v)
    for ch in range(n_ch):
        li = 2 * (ch * 16 + iota)
        idx0_v[ch, :] = plsc.load_gather(dvec_v, [li])
        idx1_v[ch, :] = plsc.load_gather(dvec_v, [li + 1])

    def start(ch):
        b = ch % 2
        t0 = w * tok + ch * ch_tok
        return (
            pltpu.async_copy(x_hbm.at[pl.ds(t0, ch_tok)], xr_v.at[b], sem_x),
            pltpu.async_copy(o_hbm.at[idx0_v.at[ch]], o0_v.at[b], sem_g0),
            pltpu.async_copy(o_hbm.at[idx1_v.at[ch]], o1_v.at[b], sem_g1),
        )

    pend = [None] * n_ch
    st = [None] * n_ch
    pend[0] = start(0)
    for ch in range(n_ch):
        b = ch % 2
        for dsc in pend[ch]:
            dsc.wait()
        if ch + 1 < n_ch:
            if ch >= 1:
                st[ch - 1].wait()
            pend[ch + 1] = start(ch + 1)
        for i in range(ch_tok):
            bi = jnp.full((16,), ch * ch_tok + i, jnp.int32)
            pv1 = plsc.load_gather(p1_v, [bi])
            pv2 = plsc.load_gather(p2_v, [bi])

            @plsc.parallel_loop(0, _D // 16, unroll=8)
            def _(j, b=b, i=i, pv1=pv1, pv2=pv2):
                sl = pl.ds(j * 16, 16)
                xr_v[b, i, sl] = (xr_v[b, i, sl] + pv1 * o0_v[b, i, sl]
                                  + pv2 * o1_v[b, i, sl])
        st[ch] = pltpu.async_copy(
            xr_v.at[b], y_hbm.at[pl.ds(w * tok + ch * ch_tok, ch_tok)],
            sem_y)
    st[n_ch - 2].wait()
    st[n_ch - 1].wait()


_combine = pl.kernel(
    _combine_body,
    out_type=jax.ShapeDtypeStruct((_N, _D), jnp.float32),
    mesh=plsc.VectorSubcoreMesh(core_axis_name="c", subcore_axis_name="s",
                                num_cores=2, num_subcores=16),
    scratch_types=[
        pltpu.VMEM((256,), jnp.int32),
        pltpu.VMEM((8, 16), jnp.int32),
        pltpu.VMEM((8, 16), jnp.int32),
        pltpu.VMEM((2, 16, _D), jnp.float32),
        pltpu.VMEM((2, 16, _D), jnp.float32),
        pltpu.VMEM((2, 16, _D), jnp.float32),
        pltpu.VMEM((128,), jnp.float32),
        pltpu.VMEM((128,), jnp.float32),
        pltpu.SemaphoreType.DMA,
        pltpu.SemaphoreType.DMA,
        pltpu.SemaphoreType.DMA,
        pltpu.SemaphoreType.DMA,
    ],
    compiler_params=pltpu.CompilerParams(needs_layout_passes=False),
)


def kernel(x, Wg, bg, ln_g, ln_b, W1, b1, W2, b2):
    b, l, d = x.shape
    xf = x.reshape(_N, _D)
    xn, e1, e2, p1, p2, gs, st, cnts = _gating(
        xf, Wg, bg.reshape(1, _E), ln_g.reshape(1, _D), ln_b.reshape(1, _D))
    xs, dest, be = _dispatch(xn, e1.reshape(_N), e2.reshape(_N),
                             cnts.reshape(32 * 16))
    o = _ffn(be, xs, W1, b1.reshape(_E, 1, 2 * _D), W2,
             b2.reshape(_E, 1, _D))
    y = _combine(xf, o, dest, p1.reshape(_N), p2.reshape(_N))

    zloss = st[0, 0] / (_N * _E)
    s1 = st[0, 1]
    s2 = st[0, 2]
    ne = _N * _E
    mean = s1 / ne
    var = (s2 - s1 * s1 / ne) / (ne - 1)
    cvloss = var / (mean * mean + 1e-10)
    return y.reshape(b, l, d), gs.reshape(_E), zloss, cvloss
